# baseline (speedup 1.0000x reference)
import functools

import jax
import jax.numpy as jnp
from jax import lax
from jax.experimental import pallas as pl
from jax.experimental.pallas import tpu as pltpu
from jax.experimental.pallas import tpu_sc as plsc

B = 16
TIN = 2048
F = 80
TOUT = 682
FOUT = 560
BLK = 32
LOOP_BLOCKS = 10

_mesh = plsc.VectorSubcoreMesh(core_axis_name="c", subcore_axis_name="s")


def _assemble(buf, obuf, row_of, nrows=BLK):
    for s in range(nrows):
        vals = []
        for d in range(7):
            j = row_of(s, d)
            for e in range(5):
                vals.append(buf[j, pl.ds(16 * e, 16)])
        for d in range(7):
            for e in range(5):
                obuf[s, pl.ds(80 * d + 16 * e, 16)] = vals[5 * d + e]


@functools.partial(
    pl.kernel,
    mesh=_mesh,
    out_type=jax.ShapeDtypeStruct((B, TOUT, FOUT), jnp.float32),
    scratch_types=[
        pltpu.VMEM((112, F), jnp.float32),
        pltpu.VMEM((112, F), jnp.float32),
        pltpu.VMEM((BLK, FOUT), jnp.float32),
        pltpu.VMEM((BLK, FOUT), jnp.float32),
        pltpu.VMEM((10, FOUT), jnp.float32),
        pltpu.SemaphoreType.DMA,
        pltpu.SemaphoreType.DMA,
        pltpu.SemaphoreType.DMA,
        pltpu.SemaphoreType.DMA,
    ],
)
def _splice(x_hbm, out_hbm, buf0, buf1, obuf0, obuf1, tailbuf, sem0, sem1,
            lsem0, lsem1):
    b = lax.axis_index("s")
    half = lax.axis_index("c")

    bufs = (buf0, buf1)
    obufs = (obuf0, obuf1)
    sems = (sem0, sem1)
    lsems = (lsem0, lsem1)
    m0 = LOOP_BLOCKS * half

    def start_load(m, slot):
        @pl.when(m == 0)
        def _first():
            pltpu.make_async_copy(
                x_hbm.at[b, pl.ds(0, 104), :],
                bufs[slot].at[pl.ds(8, 104), :],
                lsems[slot],
            ).start()

        @pl.when(m != 0)
        def _interior():
            r0 = pl.multiple_of(96 * m - 8, 8)
            pltpu.make_async_copy(
                x_hbm.at[b, pl.ds(r0, 112), :], bufs[slot], lsems[slot]
            ).start()

    start_load(m0, 0)
    start_load(m0 + 1, 1)

    def block(it, carry):
        for slot in range(2):
            m = m0 + 2 * it + slot
            buf, obuf, sem = bufs[slot], obufs[slot], sems[slot]
            @pl.when(m == 0)
            def _wait_first():
                pltpu.make_async_copy(
                    x_hbm.at[b, pl.ds(0, 104), :],
                    buf.at[pl.ds(8, 104), :],
                    lsems[slot],
                ).wait()
                for rel in (5, 6, 7):
                    for e in range(5):
                        buf[rel, pl.ds(16 * e, 16)] = buf[8, pl.ds(16 * e, 16)]

            @pl.when(m != 0)
            def _wait_interior():
                pltpu.make_async_copy(
                    x_hbm.at[b, pl.ds(0, 112), :], buf, lsems[slot]
                ).wait()

            @pl.when(it > 0)
            def _drain():
                pltpu.make_async_copy(
                    out_hbm.at[b, pl.ds(0, BLK), :], obuf, sem
                ).wait()

            _assemble(buf, obuf, lambda s, d: 3 * s + d + 5)
            pltpu.make_async_copy(
                obuf,
                out_hbm.at[b, pl.ds(pl.multiple_of(BLK * m, 8), BLK), :],
                sem,
            ).start()

            @pl.when(it < LOOP_BLOCKS // 2 - 1)
            def _prefetch():
                start_load(m + 2, slot)
        return carry

    lax.fori_loop(0, LOOP_BLOCKS // 2, block, 0)

    for slot in range(2):
        pltpu.make_async_copy(
            out_hbm.at[b, pl.ds(0, BLK), :], obufs[slot], sems[slot]
        ).wait()

    @pl.when(half == 1)
    def _tail_block():
        pltpu.sync_copy(x_hbm.at[b, pl.ds(1912, 112), :], buf0)
        _assemble(buf0, obuf0, lambda s, d: 3 * s + d + 5)
        pltpu.sync_copy(obuf0, out_hbm.at[b, pl.ds(640, BLK), :])

        pltpu.sync_copy(x_hbm.at[b, pl.ds(2008, 40), :], buf0.at[pl.ds(0, 40), :])
        _assemble(buf0, tailbuf, lambda s, d: min(3 * s + d + 5, 37), nrows=10)
        pltpu.sync_copy(tailbuf, out_hbm.at[b, pl.ds(672, 10), :])


def kernel(x):
    return _splice(x)


# speedup vs baseline: 1.0692x; 1.0692x over previous
import functools

import jax
import jax.numpy as jnp
from jax import lax
from jax.experimental import pallas as pl
from jax.experimental.pallas import tpu as pltpu
from jax.experimental.pallas import tpu_sc as plsc

B = 16
TIN = 2048
F = 80
TOUT = 682
FOUT = 560
BLK = 16
LOOP_BLOCKS = 20

_mesh = plsc.VectorSubcoreMesh(core_axis_name="c", subcore_axis_name="s")


def _assemble(buf, obuf, row_of, nrows=BLK):
    cache = {}
    for s in range(nrows):
        rows = [row_of(s, d) for d in range(7)]
        for j in rows:
            if j not in cache:
                cache[j] = [buf[j, pl.ds(16 * e, 16)] for e in range(5)]
        for d in range(7):
            for e in range(5):
                obuf[s, pl.ds(80 * d + 16 * e, 16)] = cache[rows[d]][e]
        if s + 1 < nrows:
            keep = row_of(s + 1, 0)
            cache = {j: v for j, v in cache.items() if j >= keep}


@functools.partial(
    pl.kernel,
    mesh=_mesh,
    out_type=jax.ShapeDtypeStruct((B, TOUT, FOUT), jnp.float32),
    scratch_types=[
        pltpu.VMEM((64, F), jnp.float32),
        pltpu.VMEM((64, F), jnp.float32),
        pltpu.VMEM((BLK, FOUT), jnp.float32),
        pltpu.VMEM((BLK, FOUT), jnp.float32),
        pltpu.VMEM((10, FOUT), jnp.float32),
        pltpu.SemaphoreType.DMA,
        pltpu.SemaphoreType.DMA,
        pltpu.SemaphoreType.DMA,
        pltpu.SemaphoreType.DMA,
    ],
)
def _splice(x_hbm, out_hbm, buf0, buf1, obuf0, obuf1, tailbuf, sem0, sem1,
            lsem0, lsem1):
    b = lax.axis_index("s")
    half = lax.axis_index("c")

    bufs = (buf0, buf1)
    obufs = (obuf0, obuf1)
    sems = (sem0, sem1)
    lsems = (lsem0, lsem1)
    m0 = 1 + LOOP_BLOCKS * half

    def start_load(m, slot):
        r0 = pl.multiple_of(48 * m - 8, 8)
        pltpu.make_async_copy(
            x_hbm.at[b, pl.ds(r0, 64), :], bufs[slot], lsems[slot]
        ).start()

    def wait_load(slot):
        pltpu.make_async_copy(
            x_hbm.at[b, pl.ds(0, 64), :], bufs[slot], lsems[slot]
        ).wait()

    @pl.when(half == 0)
    def _left_block():
        pltpu.sync_copy(x_hbm.at[b, pl.ds(0, 64), :], buf0)
        _assemble(buf0, obuf0, lambda s, d: max(3 * s + d - 3, 0))
        pltpu.sync_copy(obuf0, out_hbm.at[b, pl.ds(0, BLK), :])

    start_load(m0, 0)
    start_load(m0 + 1, 1)

    def block(it, carry):
        for slot in range(2):
            m = m0 + 2 * it + slot
            buf, obuf, sem = bufs[slot], obufs[slot], sems[slot]
            wait_load(slot)

            @pl.when(it > 0)
            def _drain():
                pltpu.make_async_copy(
                    out_hbm.at[b, pl.ds(0, BLK), :], obuf, sem
                ).wait()

            _assemble(buf, obuf, lambda s, d: 3 * s + d + 5)
            pltpu.make_async_copy(
                obuf,
                out_hbm.at[b, pl.ds(pl.multiple_of(BLK * m, 8), BLK), :],
                sem,
            ).start()

            @pl.when(it < LOOP_BLOCKS // 2 - 1)
            def _prefetch():
                start_load(m + 2, slot)
        return carry

    lax.fori_loop(0, LOOP_BLOCKS // 2, block, 0)

    for slot in range(2):
        pltpu.make_async_copy(
            out_hbm.at[b, pl.ds(0, BLK), :], obufs[slot], sems[slot]
        ).wait()

    @pl.when(half == 1)
    def _trailing_blocks():
        pltpu.sync_copy(x_hbm.at[b, pl.ds(1960, 64), :], buf0)
        _assemble(buf0, obuf0, lambda s, d: 3 * s + d + 5)
        pltpu.sync_copy(obuf0, out_hbm.at[b, pl.ds(656, BLK), :])

        pltpu.sync_copy(x_hbm.at[b, pl.ds(2008, 40), :], buf1.at[pl.ds(0, 40), :])
        _assemble(buf1, tailbuf, lambda s, d: min(3 * s + d + 5, 37), nrows=10)
        pltpu.sync_copy(tailbuf, out_hbm.at[b, pl.ds(672, 10), :])


def kernel(x):
    return _splice(x)


# speedup vs baseline: 1.0861x; 1.0158x over previous
import functools

import jax
import jax.numpy as jnp
from jax import lax
from jax.experimental import pallas as pl
from jax.experimental.pallas import tpu as pltpu
from jax.experimental.pallas import tpu_sc as plsc

B = 16
TIN = 2048
F = 80
TOUT = 682
FOUT = 560
BLK = 16
LOOP_BLOCKS = 20

_mesh = plsc.VectorSubcoreMesh(core_axis_name="c", subcore_axis_name="s")


def _assemble(buf, obuf, row_of, nrows=BLK):
    cache = {}
    for s in range(nrows):
        rows = [row_of(s, d) for d in range(7)]
        for j in rows:
            if j not in cache:
                cache[j] = [buf[j, pl.ds(16 * e, 16)] for e in range(5)]
        for d in range(7):
            for e in range(5):
                obuf[s, pl.ds(80 * d + 16 * e, 16)] = cache[rows[d]][e]
        if s + 1 < nrows:
            keep = row_of(s + 1, 0)
            cache = {j: v for j, v in cache.items() if j >= keep}


@functools.partial(
    pl.kernel,
    mesh=_mesh,
    out_type=jax.ShapeDtypeStruct((B, TOUT, FOUT), jnp.float32),
    scratch_types=[
        pltpu.VMEM((64, F), jnp.float32),
        pltpu.VMEM((64, F), jnp.float32),
        pltpu.VMEM((BLK, FOUT), jnp.float32),
        pltpu.VMEM((BLK, FOUT), jnp.float32),
        pltpu.VMEM((10, FOUT), jnp.float32),
        pltpu.VMEM((64, F), jnp.float32),
        pltpu.VMEM((40, F), jnp.float32),
        pltpu.SemaphoreType.DMA,
        pltpu.SemaphoreType.DMA,
        pltpu.SemaphoreType.DMA,
        pltpu.SemaphoreType.DMA,
        pltpu.SemaphoreType.DMA,
    ],
)
def _splice(x_hbm, out_hbm, buf0, buf1, obuf0, obuf1, tailbuf, bufe, buft,
            sem0, sem1, lsem0, lsem1, seme):
    b = lax.axis_index("s")
    half = lax.axis_index("c")

    bufs = (buf0, buf1)
    obufs = (obuf0, obuf1)
    sems = (sem0, sem1)
    lsems = (lsem0, lsem1)
    m0 = 1 + LOOP_BLOCKS * half

    def start_load(m, slot):
        r0 = pl.multiple_of(48 * m - 8, 8)
        pltpu.make_async_copy(
            x_hbm.at[b, pl.ds(r0, 64), :], bufs[slot], lsems[slot]
        ).start()

    def wait_load(slot):
        pltpu.make_async_copy(
            x_hbm.at[b, pl.ds(0, 64), :], bufs[slot], lsems[slot]
        ).wait()

    @pl.when(half == 0)
    def _left_block():
        pltpu.sync_copy(x_hbm.at[b, pl.ds(0, 64), :], buf0)
        _assemble(buf0, obuf0, lambda s, d: max(3 * s + d - 3, 0))
        pltpu.sync_copy(obuf0, out_hbm.at[b, pl.ds(0, BLK), :])

    @pl.when(half == 1)
    def _prefetch_trailing():
        pltpu.make_async_copy(x_hbm.at[b, pl.ds(1960, 64), :], bufe, seme).start()
        pltpu.make_async_copy(x_hbm.at[b, pl.ds(2008, 40), :], buft, seme).start()

    start_load(m0, 0)
    start_load(m0 + 1, 1)

    def block(it, carry):
        for slot in range(2):
            m = m0 + 2 * it + slot
            buf, obuf, sem = bufs[slot], obufs[slot], sems[slot]
            wait_load(slot)

            @pl.when(it > 0)
            def _drain():
                pltpu.make_async_copy(
                    out_hbm.at[b, pl.ds(0, BLK), :], obuf, sem
                ).wait()

            _assemble(buf, obuf, lambda s, d: 3 * s + d + 5)
            pltpu.make_async_copy(
                obuf,
                out_hbm.at[b, pl.ds(pl.multiple_of(BLK * m, 8), BLK), :],
                sem,
            ).start()

            @pl.when(it < LOOP_BLOCKS // 2 - 1)
            def _prefetch():
                start_load(m + 2, slot)
        return carry

    lax.fori_loop(0, LOOP_BLOCKS // 2, block, 0)

    for slot in range(2):
        pltpu.make_async_copy(
            out_hbm.at[b, pl.ds(0, BLK), :], obufs[slot], sems[slot]
        ).wait()

    @pl.when(half == 1)
    def _trailing_blocks():
        pltpu.make_async_copy(x_hbm.at[b, pl.ds(1960, 64), :], bufe, seme).wait()
        pltpu.make_async_copy(x_hbm.at[b, pl.ds(2008, 40), :], buft, seme).wait()

        _assemble(bufe, obuf0, lambda s, d: 3 * s + d + 5)
        pltpu.make_async_copy(obuf0, out_hbm.at[b, pl.ds(656, BLK), :], seme).start()

        _assemble(buft, tailbuf, lambda s, d: min(3 * s + d + 5, 37), nrows=10)
        pltpu.sync_copy(tailbuf, out_hbm.at[b, pl.ds(672, 10), :])
        pltpu.make_async_copy(obuf0, out_hbm.at[b, pl.ds(656, BLK), :], seme).wait()


def kernel(x):
    return _splice(x)
